# baseline (speedup 1.0000x reference)
import functools

import jax
import jax.numpy as jnp
from jax import lax
from jax.experimental import pallas as pl
from jax.experimental.pallas import tpu as pltpu
from jax.experimental.pallas import tpu_sc as plsc

_B, _T, _C = 16, 1024, 768
_NC, _NS = 2, 16
_NW = _NC * _NS
_ROWS_PER_W = _B * _T // _NW
_BLK = 32
_NBLK = _ROWS_PER_W // _BLK
_NBUF = 5


def _shuffle_body(x_hbm, idx_hbm, out_hbm, idx_v, *rest):
    bufs = rest[:_NBUF]
    gsems = rest[_NBUF:2 * _NBUF]
    ssems = rest[2 * _NBUF:]
    b = lax.axis_index("s")
    half = lax.axis_index("c")
    t_base = half * _ROWS_PER_W
    w_base = b * _T + t_base
    xb = x_hbm.at[pl.ds(b * _T, _T)]

    idx_cp = pltpu.async_copy(
        idx_hbm.at[pl.ds(t_base, _ROWS_PER_W)], idx_v, gsems[_NBUF - 1])

    def issue_read(g):
        buf = bufs[g % _NBUF]
        sem = gsems[g % _NBUF]
        if g % 2 == 0:
            return pltpu.async_copy(
                x_hbm.at[pl.ds(w_base + g * _BLK, _BLK)], buf, sem)
        return pltpu.async_copy(
            xb.at[idx_v.at[pl.ds(g * _BLK, _BLK)]], buf, sem)

    gs = [None] * _NBLK
    ss = [None] * _NBLK
    gs[0] = issue_read(0)
    idx_cp.wait()
    for g in range(1, _NBUF - 1):
        gs[g] = issue_read(g)
    for g in range(_NBLK):
        nx = g + _NBUF - 1
        if nx < _NBLK:
            if nx >= _NBUF:
                ss[nx - _NBUF].wait()
            gs[nx] = issue_read(nx)
        gs[g].wait()
        ss[g] = pltpu.async_copy(
            bufs[g % _NBUF],
            out_hbm.at[pl.ds(w_base + g * _BLK, _BLK)],
            ssems[g % _NBUF])
    for g in range(_NBLK - _NBUF, _NBLK):
        ss[g].wait()


_shuffle = functools.partial(
    pl.kernel,
    mesh=plsc.VectorSubcoreMesh(core_axis_name="c", subcore_axis_name="s"),
    out_type=jax.ShapeDtypeStruct((_B * _T, _C), jnp.float32),
    scratch_types=(
        [pltpu.VMEM((_ROWS_PER_W,), jnp.int32)]
        + [pltpu.VMEM((_BLK, _C), jnp.float32) for _ in range(_NBUF)]
        + [pltpu.SemaphoreType.DMA for _ in range(2 * _NBUF)]
    ),
)(_shuffle_body)


def kernel(x, forward_shuffle_idx):
    x2 = x.reshape(_B * _T, _C)
    out = _shuffle(x2, forward_shuffle_idx)
    return out.reshape(_B, _T, _C)


# speedup vs baseline: 1.0126x; 1.0126x over previous
import functools

import jax
import jax.numpy as jnp
from jax import lax
from jax.experimental import pallas as pl
from jax.experimental.pallas import tpu as pltpu
from jax.experimental.pallas import tpu_sc as plsc

_B, _T, _C = 16, 1024, 768
_NC, _NS = 2, 16
_NW = _NC * _NS
_ROWS_PER_W = _B * _T // _NW
_BLK = 32
_NBLK = _ROWS_PER_W // _BLK
_NBUF = 5


def _shuffle_body(x_hbm, gidx_hbm, out_hbm, idx_v, *rest):
    bufs = rest[:_NBUF]
    gsems = rest[_NBUF:2 * _NBUF]
    ssems = rest[2 * _NBUF:]
    b = lax.axis_index("s")
    half = lax.axis_index("c")
    w_base = (b * _NC + half) * _ROWS_PER_W

    idx_cp = pltpu.async_copy(
        gidx_hbm.at[b, pl.ds(half * _NBLK, _NBLK)], idx_v, gsems[_NBUF - 1])

    def issue_read(g):
        buf = bufs[g % _NBUF]
        sem = gsems[g % _NBUF]
        if g % 2 == 0:
            return pltpu.async_copy(
                x_hbm.at[pl.ds(w_base + g * _BLK, _BLK)], buf, sem)
        return pltpu.async_copy(x_hbm.at[idx_v.at[g]], buf, sem)

    gs = [None] * _NBLK
    ss = [None] * _NBLK
    gs[0] = issue_read(0)
    idx_cp.wait()
    for g in range(1, _NBUF - 1):
        gs[g] = issue_read(g)
    for g in range(_NBLK):
        nx = g + _NBUF - 1
        if nx < _NBLK:
            if nx >= _NBUF:
                ss[nx - _NBUF].wait()
            gs[nx] = issue_read(nx)
        gs[g].wait()
        ss[g] = pltpu.async_copy(
            bufs[g % _NBUF],
            out_hbm.at[pl.ds(w_base + g * _BLK, _BLK)],
            ssems[g % _NBUF])
    for g in range(_NBLK - _NBUF, _NBLK):
        ss[g].wait()


_shuffle = functools.partial(
    pl.kernel,
    mesh=plsc.VectorSubcoreMesh(core_axis_name="c", subcore_axis_name="s"),
    out_type=jax.ShapeDtypeStruct((_B * _T, _C), jnp.float32),
    scratch_types=(
        [pltpu.VMEM((_NBLK, _BLK), jnp.int32)]
        + [pltpu.VMEM((_BLK, _C), jnp.float32) for _ in range(_NBUF)]
        + [pltpu.SemaphoreType.DMA for _ in range(2 * _NBUF)]
    ),
)(_shuffle_body)


def kernel(x, forward_shuffle_idx):
    x2 = x.reshape(_B * _T, _C)
    gidx = (forward_shuffle_idx.reshape(_T // _BLK, _BLK)[None]
            + (_T * jnp.arange(_B, dtype=jnp.int32))[:, None, None])
    out = _shuffle(x2, gidx)
    return out.reshape(_B, _T, _C)
